# baseline (speedup 1.0000x reference)
import functools

import jax
import jax.numpy as jnp
from jax import lax
from jax.experimental import pallas as pl
from jax.experimental.pallas import tpu as pltpu
from jax.experimental.pallas import tpu_sc as plsc

_N = 10000
_E = 160000
_D = 256
_H = 128
_G = 64
_NS = 16
_NC = 2
_RT = 624
_TAIL0 = _RT * _NS
_TAIL = _N - _TAIL0
_ET = _E // _NS
_KA = 64
_CA = 159
_ETP = _CA * _KA
_JUNK = _N
_NP = _N + 16
_EPS = 1e-5
_F32 = jnp.float32


def _mesh():
    return plsc.VectorSubcoreMesh(core_axis_name="c", subcore_axis_name="s",
                                  num_cores=_NC, num_subcores=_NS)


def _build_sc_agg(interpret=False):
    out_type = jax.ShapeDtypeStruct((_NC * _N, _H), _F32)
    scratch = [
        pltpu.VMEM_SHARED((_NP, _H), _F32),
        pltpu.VMEM((_ETP,), jnp.int32),
        pltpu.VMEM((_ETP,), jnp.int32),
        pltpu.VMEM((_KA, _H), _F32),
        pltpu.VMEM((_KA, _H), _F32),
        pltpu.VMEM((_KA, _H), _F32),
        pltpu.VMEM((_KA,), jnp.int32),
        pltpu.VMEM((_KA,), jnp.int32),
        pltpu.VMEM((_KA,), jnp.int32),
        pltpu.VMEM((_KA,), jnp.int32),
        pltpu.VMEM((_KA,), jnp.int32),
        pltpu.VMEM((_KA,), jnp.int32),
    ] + [pltpu.SemaphoreType.DMA] * 6

    @functools.partial(pl.kernel, out_type=out_type, mesh=_mesh(),
                       scratch_types=scratch, interpret=interpret)
    def k(a_st, c_st, srct, dstt, p_st, aggsm, srcv, dstv, rows0, rows1,
          rows2, adj0, adj1, adj2, dc0, dc1, dc2, sg0, sg1, sg2,
          sc0, sc1, sc2):
        cid = lax.axis_index("c")
        sid = lax.axis_index("s")
        co = cid * _N
        r0 = sid * _RT

        pltpu.sync_copy(c_st.at[pl.ds(co + r0, _RT)], aggsm.at[pl.ds(r0, _RT)])

        @pl.when(sid == _NS - 1)
        def _():
            pltpu.sync_copy(c_st.at[pl.ds(_TAIL0 + co, _TAIL)],
                            aggsm.at[pl.ds(_TAIL0, _TAIL)])

        pltpu.sync_copy(srct.at[sid], srcv)
        pltpu.sync_copy(dstt.at[sid], dstv)
        plsc.subcore_barrier()

        ROWS = (rows0, rows1, rows2)
        ADJ = (adj0, adj1, adj2)
        DC = (dc0, dc1, dc2)
        SG = (sg0, sg1, sg2)
        SC = (sc0, sc1, sc2)

        def fill(x, j):
            for v in range(_KA // 16):
                sl = pl.ds(v * 16, 16)
                ADJ[x][sl] = srcv[pl.ds(j * _KA + v * 16, 16)] + co
                DC[x][sl] = dstv[pl.ds(j * _KA + v * 16, 16)]

        def gather_fire(x):
            pltpu.async_copy(a_st.at[ADJ[x]], ROWS[x], SG[x])

        def gather_wait(x):
            pltpu.make_async_copy(a_st.at[ADJ[x]], ROWS[x], SG[x]).wait()

        def scat_fire(x):
            pltpu.async_copy(ROWS[x], aggsm.at[DC[x]], SC[x], add=True)

        def scat_wait(x):
            pltpu.make_async_copy(ROWS[x], aggsm.at[DC[x]], SC[x]).wait()

        fill(0, 0)
        gather_fire(0)
        fill(1, 1)
        gather_fire(1)

        def chunk(j, x):
            nx = (x + 2) % 3

            @pl.when(1 <= j)
            def _():
                scat_wait(nx)

            @pl.when(j + 2 < _CA)
            def _():
                fill(nx, j + 2)
                gather_fire(nx)

            gather_wait(x)
            scat_fire(x)

        def step(m, carry):
            j0 = 3 * m
            for x in range(3):
                chunk(j0 + x, x)
            return carry

        lax.fori_loop(0, _CA // 3, step, 0)
        scat_wait((_CA - 1) % 3)
        plsc.subcore_barrier()

        pltpu.sync_copy(aggsm.at[pl.ds(r0, _RT)], p_st.at[pl.ds(co + r0, _RT)])

        @pl.when(sid == _NS - 1)
        def _():
            pltpu.sync_copy(aggsm.at[pl.ds(_TAIL0, _TAIL)],
                            p_st.at[pl.ds(co + _TAIL0, _TAIL)])

    return k


_RB = 1000
_NB = _N // _RB


def _build_layer_mm(act, interpret=False):
    def body(*refs):
        if act:
            (h0, h1, ss, tt, da, w1, b1, w2, w3, b3, a_st, c_st) = refs
        else:
            (h0, h1, da, w1, b1, w2, w3, b3, a_st, c_st) = refs
        x0 = h0[...]
        x1 = h1[...]
        if act:
            x0 = jnp.maximum(x0 * ss[0:1, :] + tt[0:1, :], 0.0)
            x1 = jnp.maximum(x1 * ss[1:2, :] + tt[1:2, :], 0.0)
        dot = functools.partial(jnp.dot, preferred_element_type=_F32)
        a = dot(x0, w1[0]) + dot(x1, w1[1]) + b1[...]
        b = dot(x0, w2[0]) + dot(x1, w2[1])
        c = dot(x0, w3[0]) + dot(x1, w3[1]) + b3[...]
        deg = da[:, 0:1]
        a_st[...] = a
        c_st[...] = c - deg * b

    rb0 = pl.BlockSpec((_RB, _H), lambda i, hh: (i, 0))
    rb1 = pl.BlockSpec((_RB, _H), lambda i, hh: (i + _NB, 0))
    sb = pl.BlockSpec((_NC, _H), lambda i, hh: (0, 0))
    db0 = pl.BlockSpec((_RB, _H), lambda i, hh: (i, 0))
    wb = pl.BlockSpec((_NC, _H, _H), lambda i, hh: (0, 0, hh))
    bb = pl.BlockSpec((1, _H), lambda i, hh: (0, hh))
    ob = pl.BlockSpec((_RB, _H), lambda i, hh: (hh * _NB + i, 0))
    if act:
        in_specs = [rb0, rb1, sb, sb, db0, wb, bb, wb, wb, bb]
    else:
        in_specs = [rb0, rb0, db0, wb, bb, wb, wb, bb]
    out_specs = [ob, ob]
    out_shape = [jax.ShapeDtypeStruct((_NC * _N, _H), _F32)] * 2
    return pl.pallas_call(
        body,
        grid=(_NB, _NC),
        in_specs=in_specs,
        out_specs=out_specs,
        out_shape=out_shape,
        interpret=interpret,
    )


def _build_stats(interpret=False):
    def body(p_st, g, e, s, t):
        for hh in range(_NC):
            x = p_st[pl.ds(hh * _N, _N), :]
            mean = jnp.sum(x, axis=0, keepdims=True) / _N
            msq = jnp.sum(x * x, axis=0, keepdims=True) / _N
            var = msq - mean * mean
            rstd = lax.rsqrt(var + _EPS)
            sc = g[hh:hh + 1, :] * rstd
            s[hh:hh + 1, :] = sc
            t[hh:hh + 1, :] = e[hh:hh + 1, :] - mean * sc

    out_shape = [jax.ShapeDtypeStruct((_NC, _H), _F32)] * 2
    return pl.pallas_call(body, out_shape=out_shape, interpret=interpret)


def _build_pool(interpret=False):
    def body(p_st, ss, tt, wg1, bg1, wg2, bg2, bat, out):
        h0 = jnp.maximum(p_st[pl.ds(0, _N), :] * ss[0:1, :] + tt[0:1, :], 0.0)
        h1 = jnp.maximum(p_st[pl.ds(_N, _N), :] * ss[1:2, :] + tt[1:2, :], 0.0)
        dot = functools.partial(jnp.dot, preferred_element_type=_F32)
        g1 = jnp.maximum(dot(h0, wg1[0]) + dot(h1, wg1[1]) + bg1[...], 0.0)
        g2 = jnp.maximum(
            jnp.sum(g1 * wg2[...], axis=1, keepdims=True) + bg2[0, 0], 0.0)
        b = bat[...]
        ids = lax.broadcasted_iota(jnp.int32, (_N, _G), 1)
        onehot_b = b == ids
        neg = jnp.where(onehot_b, g2, -jnp.inf)
        gmax = jnp.max(neg, axis=0, keepdims=True)
        gmax_n = jnp.sum(jnp.where(onehot_b, gmax, 0.0), axis=1,
                         keepdims=True)
        gexp = jnp.exp(g2 - gmax_n)
        onehot = onehot_b.astype(_F32)
        gsum = jnp.sum(onehot * gexp, axis=0, keepdims=True)
        gsum_n = jnp.sum(onehot * gsum, axis=1, keepdims=True)
        gate = gexp / (gsum_n + 1e-16)
        dn = (((0,), (0,)), ((), ()))
        out[:, :_H] = lax.dot_general(onehot, gate * h0, dn,
                                      preferred_element_type=_F32)
        out[:, _H:] = lax.dot_general(onehot, gate * h1, dn,
                                      preferred_element_type=_F32)

    return pl.pallas_call(
        body, out_shape=jax.ShapeDtypeStruct((_G, _D), _F32),
        interpret=interpret)


def _run(x, edge_index, batch, params, interpret=False, sc_interpret=False):
    src = edge_index[0].astype(jnp.int32)
    dst = edge_index[1].astype(jnp.int32)
    pad = _ETP - _ET
    srct = jnp.pad(src.reshape(_NS, _ET), ((0, 0), (0, pad)))
    dstt = jnp.pad(dst.reshape(_NS, _ET), ((0, 0), (0, pad)),
                   constant_values=_JUNK)

    sc_agg = _build_sc_agg(sc_interpret)
    mm_first = _build_layer_mm(False, interpret)
    mm_act = _build_layer_mm(True, interpret)
    stats = _build_stats(interpret)
    pool = _build_pool(interpret)

    ones_st = jnp.ones((_NC * _N, _H), _F32)
    zeros_st = jnp.zeros((_NC * _N, _H), _F32)
    d_st = sc_agg(ones_st, zeros_st, srct, dstt)

    def wsplit(p):
        w1 = p["W1"].T.reshape(_NC, _H, _D)
        w2 = p["W2"].T.reshape(_NC, _H, _D)
        w3 = p["W3"].T.reshape(_NC, _H, _D)
        return (w1, p["b1"].reshape(1, _D), w2, w3, p["b3"].reshape(1, _D))

    lw = [wsplit(p) for p in params["layers"]]

    x0 = x[:, :_H]
    x1 = x[:, _H:]
    w1, b1, w2, w3, b3 = lw[0]
    a_st, c_st = mm_first(x0, x1, d_st, w1, b1, w2, w3, b3)

    for l in range(6):
        p_st = sc_agg(a_st, c_st, srct, dstt)
        pr = params["layers"][l]
        g = pr["gamma"].reshape(_NC, _H)
        e = pr["beta"].reshape(_NC, _H)
        ss, tt = stats(p_st, g, e)
        if l < 5:
            w1, b1, w2, w3, b3 = lw[l + 1]
            a_st, c_st = mm_act(p_st, p_st, ss, tt, d_st,
                                w1, b1, w2, w3, b3)

    g = params["gate"]
    wg1 = g["Wg1"].T.reshape(_NC, _H, _G)
    bg1 = g["bg1"].reshape(1, _G)
    wg2 = g["Wg2"].reshape(1, _G)
    bg2 = g["bg2"].reshape(1, 1)
    bat = batch.astype(jnp.int32).reshape(_N, 1)
    return pool(p_st, ss, tt, wg1, bg1, wg2, bg2, bat)


def kernel(x, edge_index, batch, params):
    return _run(x, edge_index, batch, params)


# speedup vs baseline: 1.1912x; 1.1912x over previous
import functools

import jax
import jax.numpy as jnp
from jax import lax
from jax.experimental import pallas as pl
from jax.experimental.pallas import tpu as pltpu
from jax.experimental.pallas import tpu_sc as plsc

_N = 10000
_E = 160000
_D = 256
_H = 128
_G = 64
_NS = 16
_NC = 2
_RT = 624
_TAIL0 = _RT * _NS
_TAIL = _N - _TAIL0
_ET = _E // _NS
_KA = 80
_CA = 126
_ETP = _CA * _KA
_JUNK = _N
_NP = _N + 16
_EPS = 1e-5
_F32 = jnp.float32


def _mesh():
    return plsc.VectorSubcoreMesh(core_axis_name="c", subcore_axis_name="s",
                                  num_cores=_NC, num_subcores=_NS)


def _build_sc_agg(interpret=False):
    out_type = jax.ShapeDtypeStruct((_NC * _N, _H), _F32)
    scratch = [
        pltpu.VMEM_SHARED((_NP, _H), _F32),
        pltpu.VMEM((_ETP,), jnp.int32),
        pltpu.VMEM((_CA, _KA), jnp.int32),
        pltpu.VMEM((_KA,), jnp.int32),
        pltpu.VMEM((_KA,), jnp.int32),
        pltpu.VMEM((_KA, _H), _F32),
        pltpu.VMEM((_KA, _H), _F32),
        pltpu.SemaphoreType.DMA,
        pltpu.SemaphoreType.DMA,
    ]

    @functools.partial(pl.kernel, out_type=out_type, mesh=_mesh(),
                       scratch_types=scratch, interpret=interpret)
    def k(a_st, c_st, srct, dstt, p_st, aggsm, srcv, dstv,
          adj0, adj1, rows0, rows1, sg0, sg1):
        cid = lax.axis_index("c")
        sid = lax.axis_index("s")
        co = cid * _N
        r0 = sid * _RT

        pltpu.sync_copy(c_st.at[pl.ds(co + r0, _RT)], aggsm.at[pl.ds(r0, _RT)])

        @pl.when(sid == _NS - 1)
        def _():
            pltpu.sync_copy(c_st.at[pl.ds(_TAIL0 + co, _TAIL)],
                            aggsm.at[pl.ds(_TAIL0, _TAIL)])

        pltpu.sync_copy(srct.at[sid], srcv)
        pltpu.sync_copy(dstt.at[sid], dstv)
        plsc.subcore_barrier()

        def fill(adj, j):
            for v in range(_KA // 16):
                adj[pl.ds(v * 16, 16)] = srcv[pl.ds(j * _KA + v * 16, 16)] + co

        def gather_fire(adj, rows, sem):
            pltpu.async_copy(a_st.at[adj], rows, sem)

        def gather_wait(adj, rows, sem):
            pltpu.make_async_copy(a_st.at[adj], rows, sem).wait()

        fill(adj0, 0)
        gather_fire(adj0, rows0, sg0)

        def step(m, carry):
            j0 = 2 * m
            fill(adj1, j0 + 1)
            gather_fire(adj1, rows1, sg1)
            gather_wait(adj0, rows0, sg0)
            pltpu.sync_copy(rows0, aggsm.at[dstv.at[j0]], add=True)

            @pl.when(j0 + 2 < _CA)
            def _():
                fill(adj0, j0 + 2)
                gather_fire(adj0, rows0, sg0)

            gather_wait(adj1, rows1, sg1)
            pltpu.sync_copy(rows1, aggsm.at[dstv.at[j0 + 1]], add=True)
            return carry

        lax.fori_loop(0, _CA // 2, step, 0)
        plsc.subcore_barrier()

        pltpu.sync_copy(aggsm.at[pl.ds(r0, _RT)], p_st.at[pl.ds(co + r0, _RT)])

        @pl.when(sid == _NS - 1)
        def _():
            pltpu.sync_copy(aggsm.at[pl.ds(_TAIL0, _TAIL)],
                            p_st.at[pl.ds(co + _TAIL0, _TAIL)])

    return k


_RB = 1000
_NB = _N // _RB


def _build_layer_mm(act, interpret=False):
    def body(*refs):
        if act:
            (h0, h1, ss, tt, da, w1, b1, w2, w3, b3, a_st, c_st) = refs
        else:
            (h0, h1, da, w1, b1, w2, w3, b3, a_st, c_st) = refs
        x0 = h0[...]
        x1 = h1[...]
        if act:
            x0 = jnp.maximum(x0 * ss[0:1, :] + tt[0:1, :], 0.0)
            x1 = jnp.maximum(x1 * ss[1:2, :] + tt[1:2, :], 0.0)
        dot = functools.partial(jnp.dot, preferred_element_type=_F32)
        a = dot(x0, w1[0]) + dot(x1, w1[1]) + b1[...]
        b = dot(x0, w2[0]) + dot(x1, w2[1])
        c = dot(x0, w3[0]) + dot(x1, w3[1]) + b3[...]
        deg = da[:, 0:1]
        a_st[...] = a
        c_st[...] = c - deg * b

    rb0 = pl.BlockSpec((_RB, _H), lambda i, hh: (i, 0))
    rb1 = pl.BlockSpec((_RB, _H), lambda i, hh: (i + _NB, 0))
    sb = pl.BlockSpec((_NC, _H), lambda i, hh: (0, 0))
    db0 = pl.BlockSpec((_RB, _H), lambda i, hh: (i, 0))
    wb = pl.BlockSpec((_NC, _H, _H), lambda i, hh: (0, 0, hh))
    bb = pl.BlockSpec((1, _H), lambda i, hh: (0, hh))
    ob = pl.BlockSpec((_RB, _H), lambda i, hh: (hh * _NB + i, 0))
    if act:
        in_specs = [rb0, rb1, sb, sb, db0, wb, bb, wb, wb, bb]
    else:
        in_specs = [rb0, rb0, db0, wb, bb, wb, wb, bb]
    out_specs = [ob, ob]
    out_shape = [jax.ShapeDtypeStruct((_NC * _N, _H), _F32)] * 2
    return pl.pallas_call(
        body,
        grid=(_NB, _NC),
        in_specs=in_specs,
        out_specs=out_specs,
        out_shape=out_shape,
        interpret=interpret,
    )


def _build_stats(interpret=False):
    def body(p_st, g, e, s, t):
        for hh in range(_NC):
            x = p_st[pl.ds(hh * _N, _N), :]
            mean = jnp.sum(x, axis=0, keepdims=True) / _N
            msq = jnp.sum(x * x, axis=0, keepdims=True) / _N
            var = msq - mean * mean
            rstd = lax.rsqrt(var + _EPS)
            sc = g[hh:hh + 1, :] * rstd
            s[hh:hh + 1, :] = sc
            t[hh:hh + 1, :] = e[hh:hh + 1, :] - mean * sc

    out_shape = [jax.ShapeDtypeStruct((_NC, _H), _F32)] * 2
    return pl.pallas_call(body, out_shape=out_shape, interpret=interpret)


def _build_pool(interpret=False):
    def body(p_st, ss, tt, wg1, bg1, wg2, bg2, bat, out):
        h0 = jnp.maximum(p_st[pl.ds(0, _N), :] * ss[0:1, :] + tt[0:1, :], 0.0)
        h1 = jnp.maximum(p_st[pl.ds(_N, _N), :] * ss[1:2, :] + tt[1:2, :], 0.0)
        dot = functools.partial(jnp.dot, preferred_element_type=_F32)
        g1 = jnp.maximum(dot(h0, wg1[0]) + dot(h1, wg1[1]) + bg1[...], 0.0)
        g2 = jnp.maximum(
            jnp.sum(g1 * wg2[...], axis=1, keepdims=True) + bg2[0, 0], 0.0)
        b = bat[...]
        ids = lax.broadcasted_iota(jnp.int32, (_N, _G), 1)
        onehot_b = b == ids
        neg = jnp.where(onehot_b, g2, -jnp.inf)
        gmax = jnp.max(neg, axis=0, keepdims=True)
        gmax_n = jnp.sum(jnp.where(onehot_b, gmax, 0.0), axis=1,
                         keepdims=True)
        gexp = jnp.exp(g2 - gmax_n)
        onehot = onehot_b.astype(_F32)
        gsum = jnp.sum(onehot * gexp, axis=0, keepdims=True)
        gsum_n = jnp.sum(onehot * gsum, axis=1, keepdims=True)
        gate = gexp / (gsum_n + 1e-16)
        dn = (((0,), (0,)), ((), ()))
        out[:, :_H] = lax.dot_general(onehot, gate * h0, dn,
                                      preferred_element_type=_F32)
        out[:, _H:] = lax.dot_general(onehot, gate * h1, dn,
                                      preferred_element_type=_F32)

    return pl.pallas_call(
        body, out_shape=jax.ShapeDtypeStruct((_G, _D), _F32),
        interpret=interpret)


def _run(x, edge_index, batch, params, interpret=False, sc_interpret=False):
    src = edge_index[0].astype(jnp.int32)
    dst = edge_index[1].astype(jnp.int32)
    pad = _ETP - _ET
    srct = jnp.pad(src.reshape(_NS, _ET), ((0, 0), (0, pad)))
    dstt = jnp.pad(dst.reshape(_NS, _ET), ((0, 0), (0, pad)),
                   constant_values=_JUNK).reshape(_NS, _CA, _KA)

    sc_agg = _build_sc_agg(sc_interpret)
    mm_first = _build_layer_mm(False, interpret)
    mm_act = _build_layer_mm(True, interpret)
    stats = _build_stats(interpret)
    pool = _build_pool(interpret)

    ones_st = jnp.ones((_NC * _N, _H), _F32)
    zeros_st = jnp.zeros((_NC * _N, _H), _F32)
    d_st = sc_agg(ones_st, zeros_st, srct, dstt)

    def wsplit(p):
        w1 = p["W1"].T.reshape(_NC, _H, _D)
        w2 = p["W2"].T.reshape(_NC, _H, _D)
        w3 = p["W3"].T.reshape(_NC, _H, _D)
        return (w1, p["b1"].reshape(1, _D), w2, w3, p["b3"].reshape(1, _D))

    lw = [wsplit(p) for p in params["layers"]]

    x0 = x[:, :_H]
    x1 = x[:, _H:]
    w1, b1, w2, w3, b3 = lw[0]
    a_st, c_st = mm_first(x0, x1, d_st, w1, b1, w2, w3, b3)

    for l in range(6):
        p_st = sc_agg(a_st, c_st, srct, dstt)
        pr = params["layers"][l]
        g = pr["gamma"].reshape(_NC, _H)
        e = pr["beta"].reshape(_NC, _H)
        ss, tt = stats(p_st, g, e)
        if l < 5:
            w1, b1, w2, w3, b3 = lw[l + 1]
            a_st, c_st = mm_act(p_st, p_st, ss, tt, d_st,
                                w1, b1, w2, w3, b3)

    g = params["gate"]
    wg1 = g["Wg1"].T.reshape(_NC, _H, _G)
    bg1 = g["bg1"].reshape(1, _G)
    wg2 = g["Wg2"].reshape(1, _G)
    bg2 = g["bg2"].reshape(1, 1)
    bat = batch.astype(jnp.int32).reshape(_N, 1)
    return pool(p_st, ss, tt, wg1, bg1, wg2, bg2, bat)


def kernel(x, edge_index, batch, params):
    return _run(x, edge_index, batch, params)
